# baseline (speedup 1.0000x reference)
import functools

import numpy as np
import jax
import jax.numpy as jnp
from jax import lax
from jax.experimental import pallas as pl
from jax.experimental.pallas import tpu as pltpu
from jax.experimental.pallas import tpu_sc as plsc

N = 512
E = 32768
H = 512
P = 4

NS = 16
LANES = 16

GD = 4
GE = 4
STRIP = N // GD
SWORDS = STRIP * N
EG = E // GE
SCAN_IT = EG // LANES


def _dropout_scales():
    dk = jax.random.split(jax.random.key(42), 3)
    return [
        np.asarray(jax.random.bernoulli(k, 0.5, (N, H)), dtype=np.float32) * 2.0
        for k in dk
    ]


_M1, _M2, _M3 = _dropout_scales()

_R = np.equal.outer(np.arange(H), np.arange(H * P) // P).astype(np.float32)
_SEL = (np.arange(H * P)[:, None] % P == np.arange(P)[None, :]).astype(np.float32)


@functools.cache
def _sc_edge_counts_fn():
    mesh = plsc.VectorSubcoreMesh(
        core_axis_name="c", subcore_axis_name="s",
        num_cores=1, num_subcores=NS)
    return functools.partial(
        pl.kernel,
        out_type=jax.ShapeDtypeStruct((GE, N * N), jnp.float32),
        mesh=mesh,
        scratch_types=[
            pltpu.VMEM((EG,), jnp.int32),
            pltpu.VMEM((EG,), jnp.int32),
            pltpu.VMEM((SWORDS,), jnp.float32),
        ],
        compiler_params=pltpu.CompilerParams(needs_layout_passes=False),
    )(_sc_edge_counts_body)


def _sc_edge_counts_body(edges_hbm, zeros_hbm, out_hbm, src_v, dst_v, cnt_v):
    sid = lax.axis_index("s")
    grp = sid // GD
    stp = sid % GD
    dbase = stp * STRIP

    pltpu.sync_copy(zeros_hbm, cnt_v)
    pltpu.sync_copy(edges_hbm.at[0, pl.ds(grp * EG, EG)], src_v)
    pltpu.sync_copy(edges_hbm.at[1, pl.ds(grp * EG, EG)], dst_v)

    ones16 = jnp.ones((LANES,), jnp.float32)

    def edge_body(i, carry):
        s = src_v[pl.ds(i * LANES, LANES)]
        d = dst_v[pl.ds(i * LANES, LANES)]
        rel = d - dbase
        m = (rel >= 0) & (rel < STRIP)
        idx = jnp.where(m, rel * N + s, 0)
        plsc.addupdate_scatter(cnt_v, [idx], ones16, mask=m)
        return carry

    lax.fori_loop(0, SCAN_IT, edge_body, 0)

    pltpu.sync_copy(cnt_v, out_hbm.at[grp, pl.ds(dbase * N, SWORDS)])


def _tc_body(cnt_ref, x_ref, w1_ref, w2_ref, w3_ref, wq_ref,
             b1_ref, b2_ref, b3_ref, bq_ref, m1_ref, m2_ref, m3_ref,
             r_ref, sel_ref, out_ref):
    c = cnt_ref[0]
    for g in range(1, GE):
        c = c + cnt_ref[g]
    row = lax.broadcasted_iota(jnp.int32, (N, N), 0)
    col = lax.broadcasted_iota(jnp.int32, (N, N), 1)
    m = c + jnp.where(row == col, jnp.float32(1.0), jnp.float32(0.0))
    deg = jnp.sum(m, axis=1, keepdims=True)
    dinv = lax.rsqrt(jnp.maximum(deg, 1.0))

    def dot(a, b):
        return lax.dot_general(a, b, (((1,), (0,)), ((), ())),
                               precision=lax.Precision.HIGHEST,
                               preferred_element_type=jnp.float32)

    h = x_ref[...] * w1_ref[...]
    h = jnp.maximum(dinv * dot(m, dinv * h) + b1_ref[...], 0.0) * m1_ref[...]
    h = dot(h, w2_ref[...])
    h = jnp.maximum(dinv * dot(m, dinv * h) + b2_ref[...], 0.0) * m2_ref[...]
    h = dot(h, w3_ref[...])
    h = jnp.maximum(dinv * dot(m, dinv * h) + b3_ref[...], 0.0) * m3_ref[...]

    hx = dot(h, r_ref[...])
    out_ref[...] = dot(hx * wq_ref[...], sel_ref[...]) + bq_ref[...]


def kernel(x_list, edge_index, W1, b1, W2, b2, W3, b3, Wq, bq):
    zeros = jnp.zeros((SWORDS,), jnp.float32)
    cnt = _sc_edge_counts_fn()(edge_index.astype(jnp.int32), zeros)
    q = pl.pallas_call(
        _tc_body,
        out_shape=jax.ShapeDtypeStruct((N, P), jnp.float32),
    )(cnt.reshape(GE, N, N), x_list, W1, W2, W3, Wq.reshape(N, H * P),
      b1.reshape(1, H), b2.reshape(1, H), b3.reshape(1, H), bq,
      _M1, _M2, _M3, _R, _SEL)
    return q


# speedup vs baseline: 1.2536x; 1.2536x over previous
import functools

import numpy as np
import jax
import jax.numpy as jnp
from jax import lax
from jax.experimental import pallas as pl
from jax.experimental.pallas import tpu as pltpu
from jax.experimental.pallas import tpu_sc as plsc

N = 512
E = 32768
H = 512
P = 4

NC = 2
NS = 16
NW = NC * NS
LANES = 16

GD = 4
GE = 4
STRIP = N // GD
SWORDS = STRIP * N
EG = E // GE
SCAN_IT = EG // LANES


def _dropout_scales():
    dk = jax.random.split(jax.random.key(42), 3)
    return [
        np.asarray(jax.random.bernoulli(k, 0.5, (N, H)), dtype=np.float32) * 2.0
        for k in dk
    ]


_M1, _M2, _M3 = _dropout_scales()

_R = np.equal.outer(np.arange(H), np.arange(H * P) // P).astype(np.float32)
_SEL = (np.arange(H * P)[:, None] % P == np.arange(P)[None, :]).astype(np.float32)


@functools.cache
def _sc_edge_counts_fn():
    mesh = plsc.VectorSubcoreMesh(
        core_axis_name="c", subcore_axis_name="s",
        num_cores=1, num_subcores=NS)
    return functools.partial(
        pl.kernel,
        out_type=jax.ShapeDtypeStruct((GE, N * N), jnp.float32),
        mesh=mesh,
        scratch_types=[
            pltpu.VMEM((EG,), jnp.int32),
            pltpu.VMEM((EG,), jnp.int32),
            pltpu.VMEM((SWORDS,), jnp.float32),
        ],
        compiler_params=pltpu.CompilerParams(needs_layout_passes=False),
    )(_sc_edge_counts_body)


def _sc_edge_counts_body(edges_hbm, zeros_hbm, out_hbm, src_v, dst_v, cnt_v):
    sid = lax.axis_index("s")
    grp = sid // GD
    stp = sid % GD
    dbase = stp * STRIP

    pltpu.sync_copy(zeros_hbm.at[sid], cnt_v)
    pltpu.sync_copy(edges_hbm.at[0, pl.ds(grp * EG, EG)], src_v)
    pltpu.sync_copy(edges_hbm.at[1, pl.ds(grp * EG, EG)], dst_v)

    ones16 = jnp.ones((LANES,), jnp.float32)

    def edge_body(i, carry):
        s = src_v[pl.ds(i * LANES, LANES)]
        d = dst_v[pl.ds(i * LANES, LANES)]
        rel = d - dbase
        m = (rel >= 0) & (rel < STRIP)
        idx = jnp.where(m, rel * N + s, 0)
        plsc.addupdate_scatter(cnt_v, [idx], ones16, mask=m)
        return carry

    lax.fori_loop(0, SCAN_IT, edge_body, 0)

    pltpu.sync_copy(cnt_v, out_hbm.at[grp, pl.ds(dbase * N, SWORDS)])


def _tc_body(cnt_ref, x_ref, w1_ref, w2_ref, w3_ref, wq_ref,
             b1_ref, b2_ref, b3_ref, bq_ref, m1_ref, m2_ref, m3_ref,
             r_ref, sel_ref, out_ref):
    c = cnt_ref[0]
    for g in range(1, GE):
        c = c + cnt_ref[g]
    row = lax.broadcasted_iota(jnp.int32, (N, N), 0)
    col = lax.broadcasted_iota(jnp.int32, (N, N), 1)
    m = c + jnp.where(row == col, jnp.float32(1.0), jnp.float32(0.0))
    deg = jnp.sum(m, axis=1, keepdims=True)
    dinv = lax.rsqrt(jnp.maximum(deg, 1.0))

    def dot(a, b):
        return lax.dot_general(a, b, (((1,), (0,)), ((), ())),
                               precision=lax.Precision.HIGHEST,
                               preferred_element_type=jnp.float32)

    h = x_ref[...] * w1_ref[...]
    h = jnp.maximum(dinv * dot(m, dinv * h) + b1_ref[...], 0.0) * m1_ref[...]
    h = dot(h, w2_ref[...])
    h = jnp.maximum(dinv * dot(m, dinv * h) + b2_ref[...], 0.0) * m2_ref[...]
    h = dot(h, w3_ref[...])
    h = jnp.maximum(dinv * dot(m, dinv * h) + b3_ref[...], 0.0) * m3_ref[...]

    def dot_fast(a, b):
        return lax.dot_general(a, b, (((1,), (0,)), ((), ())),
                               preferred_element_type=jnp.float32)

    hx = dot_fast(h, r_ref[...])
    out_ref[...] = dot_fast(hx * wq_ref[...], sel_ref[...]) + bq_ref[...]


def kernel(x_list, edge_index, W1, b1, W2, b2, W3, b3, Wq, bq):
    zeros = jnp.zeros((NS, SWORDS), jnp.float32)
    cnt = _sc_edge_counts_fn()(edge_index.astype(jnp.int32), zeros)
    q = pl.pallas_call(
        _tc_body,
        out_shape=jax.ShapeDtypeStruct((N, P), jnp.float32),
    )(cnt.reshape(GE, N, N), x_list, W1, W2, W3, Wq.reshape(N, H * P),
      b1.reshape(1, H), b2.reshape(1, H), b3.reshape(1, H), bq,
      _M1, _M2, _M3, _R, _SEL)
    return q


# speedup vs baseline: 1.4367x; 1.1461x over previous
import functools

import numpy as np
import jax
import jax.numpy as jnp
from jax import lax
from jax.experimental import pallas as pl
from jax.experimental.pallas import tpu as pltpu
from jax.experimental.pallas import tpu_sc as plsc

N = 512
E = 32768
H = 512
P = 4

NC = 2
NS = 16
NW = NC * NS
LANES = 16

GD = 4
GE = 4
STRIP = N // GD
SWORDS = STRIP * N
EG = E // GE
SCAN_IT = EG // LANES


def _dropout_scales():
    dk = jax.random.split(jax.random.key(42), 3)
    return [
        np.asarray(jax.random.bernoulli(k, 0.5, (N, H)), dtype=np.float32) * 2.0
        for k in dk
    ]


_M1, _M2, _M3 = _dropout_scales()

_R = np.equal.outer(np.arange(H), np.arange(H * P) // P).astype(np.float32)
_SEL = (np.arange(H * P)[:, None] % P == np.arange(P)[None, :]).astype(np.float32)


@functools.cache
def _sc_edge_counts_fn():
    mesh = plsc.VectorSubcoreMesh(
        core_axis_name="c", subcore_axis_name="s",
        num_cores=1, num_subcores=NS)
    return functools.partial(
        pl.kernel,
        out_type=jax.ShapeDtypeStruct((GE, N * N), jnp.float32),
        mesh=mesh,
        scratch_types=[
            pltpu.VMEM((EG,), jnp.int32),
            pltpu.VMEM((EG,), jnp.int32),
            pltpu.VMEM((SWORDS,), jnp.float32),
        ],
        compiler_params=pltpu.CompilerParams(needs_layout_passes=False),
    )(_sc_edge_counts_body)


def _sc_edge_counts_body(edges_hbm, zeros_hbm, out_hbm, src_v, dst_v, cnt_v):
    sid = lax.axis_index("s")
    grp = sid // GD
    stp = sid % GD
    dbase = stp * STRIP

    pltpu.sync_copy(zeros_hbm.at[sid], cnt_v)
    pltpu.sync_copy(edges_hbm.at[0, pl.ds(grp * EG, EG)], src_v)
    pltpu.sync_copy(edges_hbm.at[1, pl.ds(grp * EG, EG)], dst_v)

    ones16 = jnp.ones((LANES,), jnp.float32)

    def edge_body(i, carry):
        s = src_v[pl.ds(i * LANES, LANES)]
        d = dst_v[pl.ds(i * LANES, LANES)]
        rel = d - dbase
        m = (rel >= 0) & (rel < STRIP)
        idx = jnp.where(m, rel * N + s, 0)
        plsc.addupdate_scatter(cnt_v, [idx], ones16, mask=m)
        return carry

    lax.fori_loop(0, SCAN_IT, edge_body, 0)

    pltpu.sync_copy(cnt_v, out_hbm.at[grp, pl.ds(dbase * N, SWORDS)])


def _tc_body(cnt_ref, x_ref, w1_ref, w2_ref, w3_ref, wq_ref,
             b1_ref, b2_ref, b3_ref, bq_ref, m1_ref, m2_ref, m3_ref,
             r_ref, sel_ref, out_ref):
    c = cnt_ref[0]
    for g in range(1, GE):
        c = c + cnt_ref[g]
    row = lax.broadcasted_iota(jnp.int32, (N, N), 0)
    col = lax.broadcasted_iota(jnp.int32, (N, N), 1)
    m = c + jnp.where(row == col, jnp.float32(1.0), jnp.float32(0.0))
    deg = jnp.sum(m, axis=1, keepdims=True)
    dinv = lax.rsqrt(jnp.maximum(deg, 1.0))

    def dot(a, b):
        return lax.dot_general(a, b, (((1,), (0,)), ((), ())),
                               preferred_element_type=jnp.float32)

    h = x_ref[...] * w1_ref[...]
    h = jnp.maximum(dinv * dot(m, dinv * h) + b1_ref[...], 0.0) * m1_ref[...]
    h = dot(h, w2_ref[...])
    h = jnp.maximum(dinv * dot(m, dinv * h) + b2_ref[...], 0.0) * m2_ref[...]
    h = dot(h, w3_ref[...])
    h = jnp.maximum(dinv * dot(m, dinv * h) + b3_ref[...], 0.0) * m3_ref[...]

    hx = dot(h, r_ref[...])
    out_ref[...] = dot(hx * wq_ref[...], sel_ref[...]) + bq_ref[...]


def kernel(x_list, edge_index, W1, b1, W2, b2, W3, b3, Wq, bq):
    zeros = jnp.zeros((NS, SWORDS), jnp.float32)
    cnt = _sc_edge_counts_fn()(edge_index.astype(jnp.int32), zeros)
    q = pl.pallas_call(
        _tc_body,
        out_shape=jax.ShapeDtypeStruct((N, P), jnp.float32),
    )(cnt.reshape(GE, N, N), x_list, W1, W2, W3, Wq.reshape(N, H * P),
      b1.reshape(1, H), b2.reshape(1, H), b3.reshape(1, H), bq,
      _M1, _M2, _M3, _R, _SEL)
    return q
